# Initial kernel scaffold; baseline (speedup 1.0000x reference)
import jax
import jax.numpy as jnp
from jax.experimental import pallas as pl


def kernel(x, edge_index, edge_weight, W_gcn, b_gcn, gamma, beta, W1, b1, W2, b2):
    raise NotImplementedError("write your pallas kernel here")


# speedup vs baseline: 28.9783x; 28.9783x over previous
import functools

import jax
import jax.numpy as jnp
from jax import lax
from jax.experimental import pallas as pl
from jax.experimental.pallas import tpu as pltpu
from jax.experimental.pallas import tpu_sc as plsc

N = 10000
E = 320000
D_IN = 128
H1 = 64
ROW = 80
NROWS = E // ROW
NC = 2
NS = 16
NW = NC * NS
RPW = NROWS // NW
NPAD = 10240

_sc_mesh = plsc.VectorSubcoreMesh(core_axis_name="c", subcore_axis_name="s")


@functools.partial(
    pl.kernel,
    out_type=jax.ShapeDtypeStruct((NC, NPAD), jnp.float32),
    mesh=_sc_mesh,
    scratch_types=[
        pltpu.VMEM((640,), jnp.float32),
        pltpu.VMEM((RPW, ROW), jnp.int32),
        pltpu.VMEM((RPW, ROW), jnp.float32),
        pltpu.VMEM_SHARED((NPAD,), jnp.float32),
    ],
)
def _deg_kernel(dst_hbm, ew_hbm, out_hbm, zbuf, dstv, ewv, acc):
    cid = lax.axis_index("c")
    sid = lax.axis_index("s")
    wid = sid * NC + cid
    zeros = jnp.zeros((16,), jnp.float32)
    for i in range(40):
        zbuf[pl.ds(i * 16, 16)] = zeros
    pltpu.sync_copy(zbuf, acc.at[pl.ds(sid * 640, 640)])
    plsc.subcore_barrier()
    pltpu.sync_copy(dst_hbm.at[wid], dstv)
    pltpu.sync_copy(ew_hbm.at[wid], ewv)

    def body(j, carry):
        pltpu.sync_copy(ewv.at[j], acc.at[dstv.at[j]], add=True)
        return carry

    lax.fori_loop(0, RPW, body, 0)
    plsc.subcore_barrier()

    @pl.when(sid == 0)
    def _():
        pltpu.sync_copy(acc, out_hbm.at[cid])


def _g_body(x_ref, wt_ref, deg_ref, g_ref):
    dsum = deg_ref[0] + deg_ref[1] + 1.0
    dinv = jnp.where(dsum > 0, lax.rsqrt(dsum), 0.0)
    h = jnp.dot(x_ref[...], wt_ref[...], preferred_element_type=jnp.float32)
    g_ref[...] = dinv[:N] * h


_g_call = pl.pallas_call(
    _g_body,
    out_shape=jax.ShapeDtypeStruct((N, H1), jnp.float32),
)


@functools.partial(
    pl.kernel,
    out_type=jax.ShapeDtypeStruct((NC, NPAD, H1), jnp.float32),
    mesh=_sc_mesh,
    compiler_params=pltpu.CompilerParams(use_tc_tiling_on_sc=False),
    scratch_types=[
        pltpu.VMEM((64, H1), jnp.float32),
        pltpu.VMEM((RPW, ROW), jnp.int32),
        pltpu.VMEM((RPW, ROW), jnp.int32),
        pltpu.VMEM((RPW, ROW), jnp.float32),
        pltpu.VMEM((ROW, H1), jnp.float32),
        pltpu.VMEM_SHARED((NPAD, H1), jnp.float32),
        pltpu.SemaphoreType.DMA,
    ],
)
def _agg_kernel(src_hbm, dst_hbm, ew_hbm, g_hbm, out_hbm,
                zb, srcv, dstv, ewv, rows, acc, sem):
    cid = lax.axis_index("c")
    sid = lax.axis_index("s")
    wid = sid * NC + cid
    zeros = jnp.zeros((16,), jnp.float32)
    for r in range(64):
        for k in range(H1 // 16):
            zb[r, pl.ds(k * 16, 16)] = zeros
    for k in range(10):
        pltpu.sync_copy(zb, acc.at[pl.ds(sid * 640 + k * 64, 64)])
    plsc.subcore_barrier()
    pltpu.sync_copy(src_hbm.at[wid], srcv)
    pltpu.sync_copy(dst_hbm.at[wid], dstv)
    pltpu.sync_copy(ew_hbm.at[wid], ewv)

    def body(j, carry):
        pltpu.async_copy(g_hbm.at[srcv.at[j]], rows, sem).wait()
        for grp in range(ROW // 16):
            wv = ewv[j, pl.ds(grp * 16, 16)]
            for e16 in range(16):
                e = grp * 16 + e16
                w = wv[e16]
                for k in range(H1 // 16):
                    sl = pl.ds(k * 16, 16)
                    rows[e, sl] = rows[e, sl] * w
        pltpu.sync_copy(rows, acc.at[dstv.at[j]], add=True)
        return carry

    lax.fori_loop(0, RPW, body, 0)
    plsc.subcore_barrier()
    pltpu.sync_copy(acc.at[pl.ds(sid * 640, 640)],
                    out_hbm.at[cid, pl.ds(sid * 640, 640)])


def _head_body(part_ref, g_ref, deg_ref, bg_ref, gam_ref, bet_ref,
               w1t_ref, b1_ref, w2t_ref, b2_ref, out_ref):
    dsum = deg_ref[0] + deg_ref[1] + 1.0
    dinv = jnp.where(dsum > 0, lax.rsqrt(dsum), 0.0)[:N]
    p = part_ref[0, :N, :] + part_ref[1, :N, :]
    agg = dinv * (p + g_ref[...])
    t = jnp.maximum(agg + bg_ref[...], 0.0)
    mu = jnp.mean(t, axis=1, keepdims=True)
    d = t - mu
    var = jnp.mean(d * d, axis=1, keepdims=True)
    ln = d * lax.rsqrt(var + 1e-5) * gam_ref[...] + bet_ref[...]
    pooled = jnp.sum(ln, axis=0, keepdims=True) * (1.0 / N)
    z = jnp.maximum(
        jnp.dot(pooled, w1t_ref[...], preferred_element_type=jnp.float32)
        + b1_ref[...], 0.0)
    out_ref[...] = (jnp.dot(z, w2t_ref[...], preferred_element_type=jnp.float32)
                    + b2_ref[...])


def _make_head(a_dim):
    return pl.pallas_call(
        _head_body,
        out_shape=jax.ShapeDtypeStruct((1, a_dim), jnp.float32),
    )


def kernel(x, edge_index, edge_weight, W_gcn, b_gcn, gamma, beta, W1, b1, W2, b2):
    src2 = edge_index[0].reshape(NW, RPW, ROW)
    dst2 = edge_index[1].reshape(NW, RPW, ROW)
    ew2 = edge_weight.reshape(NW, RPW, ROW)

    deg_p = _deg_kernel(dst2, ew2)
    deg3 = deg_p.reshape(NC, NPAD, 1)
    g = _g_call(x, W_gcn.T, deg3)
    part = _agg_kernel(src2, dst2, ew2, g)
    head = _make_head(W2.shape[0])
    return head(part, g, deg3, b_gcn.reshape(1, H1), gamma.reshape(1, H1),
                beta.reshape(1, H1), W1.T, b1.reshape(1, -1), W2.T,
                b2.reshape(1, -1))


# speedup vs baseline: 43.0176x; 1.4845x over previous
import functools

import jax
import jax.numpy as jnp
from jax import lax
from jax.experimental import pallas as pl
from jax.experimental.pallas import tpu as pltpu
from jax.experimental.pallas import tpu_sc as plsc

N = 10000
E = 320000
D_IN = 128
H1 = 64
ROW = 100
NC = 2
NS = 16
NW = NC * NS
RPW = E // (NW * ROW)
NPAD = 10240

_sc_mesh = plsc.VectorSubcoreMesh(core_axis_name="c", subcore_axis_name="s")
_sc_params = pltpu.CompilerParams(use_tc_tiling_on_sc=False)


@functools.partial(
    pl.kernel,
    out_type=jax.ShapeDtypeStruct((NC, NPAD), jnp.float32),
    mesh=_sc_mesh,
    compiler_params=_sc_params,
    scratch_types=[
        pltpu.VMEM((640,), jnp.float32),
        pltpu.VMEM((RPW, ROW), jnp.int32),
        pltpu.VMEM((RPW, ROW), jnp.float32),
        pltpu.VMEM_SHARED((NPAD,), jnp.float32),
        pltpu.SemaphoreType.DMA,
    ],
)
def _deg_kernel(dst_hbm, ew_hbm, out_hbm, zbuf, dstv, ewv, acc, sem):
    cid = lax.axis_index("c")
    sid = lax.axis_index("s")
    wid = sid * NC + cid
    zeros = jnp.zeros((16,), jnp.float32)
    for i in range(40):
        zbuf[pl.ds(i * 16, 16)] = zeros
    pltpu.sync_copy(zbuf, acc.at[pl.ds(sid * 640, 640)])
    plsc.subcore_barrier()
    pltpu.sync_copy(dst_hbm.at[wid], dstv)
    pltpu.sync_copy(ew_hbm.at[wid], ewv)

    def fire(j, carry):
        pltpu.async_copy(ewv.at[j], acc.at[dstv.at[j]], sem, add=True)
        return carry

    lax.fori_loop(0, RPW, fire, 0)

    def drain(j, carry):
        pltpu.make_async_copy(ewv.at[j], acc.at[dstv.at[j]], sem).wait()
        return carry

    lax.fori_loop(0, RPW, drain, 0)
    plsc.subcore_barrier()

    @pl.when(sid == 0)
    def _():
        pltpu.sync_copy(acc, out_hbm.at[cid])


def _g_body(x_ref, wt_ref, deg_ref, g_ref):
    dsum = deg_ref[0] + deg_ref[1] + 1.0
    dinv = jnp.where(dsum > 0, lax.rsqrt(dsum), 0.0)
    h = jnp.dot(x_ref[...], wt_ref[...], preferred_element_type=jnp.float32)
    g_ref[...] = dinv[:N] * h


_g_call = pl.pallas_call(
    _g_body,
    out_shape=jax.ShapeDtypeStruct((N, H1), jnp.float32),
)


def _scale_rows(buf, ewv, j):
    offs = [0, 16, 32, 48, 64, 80]
    for grp, off in enumerate(offs):
        wv = ewv[j, pl.ds(off, 16)]
        for e16 in range(16):
            e = off + e16
            w = wv[e16]
            for k in range(H1 // 16):
                sl = pl.ds(k * 16, 16)
                buf[e, sl] = buf[e, sl] * w
    wv = ewv[j, pl.ds(84, 16)]
    for e in range(96, ROW):
        w = wv[e - 84]
        for k in range(H1 // 16):
            sl = pl.ds(k * 16, 16)
            buf[e, sl] = buf[e, sl] * w


@functools.partial(
    pl.kernel,
    out_type=jax.ShapeDtypeStruct((NC, NPAD, H1), jnp.float32),
    mesh=_sc_mesh,
    compiler_params=_sc_params,
    scratch_types=[
        pltpu.VMEM((64, H1), jnp.float32),
        pltpu.VMEM((RPW, ROW), jnp.int32),
        pltpu.VMEM((RPW, ROW), jnp.int32),
        pltpu.VMEM((RPW, ROW), jnp.float32),
        pltpu.VMEM((ROW, H1), jnp.float32),
        pltpu.VMEM((ROW, H1), jnp.float32),
        pltpu.VMEM_SHARED((NPAD, H1), jnp.float32),
        pltpu.SemaphoreType.DMA,
        pltpu.SemaphoreType.DMA,
    ],
)
def _agg_kernel(src_hbm, dst_hbm, ew_hbm, g_hbm, out_hbm,
                zb, srcv, dstv, ewv, buf_a, buf_b, acc, sem_a, sem_b):
    cid = lax.axis_index("c")
    sid = lax.axis_index("s")
    wid = sid * NC + cid
    zeros = jnp.zeros((16,), jnp.float32)
    for r in range(64):
        for k in range(H1 // 16):
            zb[r, pl.ds(k * 16, 16)] = zeros
    for k in range(10):
        pltpu.sync_copy(zb, acc.at[pl.ds(sid * 640 + k * 64, 64)])
    plsc.subcore_barrier()
    pltpu.sync_copy(src_hbm.at[wid], srcv)
    pltpu.sync_copy(dst_hbm.at[wid], dstv)
    pltpu.sync_copy(ew_hbm.at[wid], ewv)

    pltpu.async_copy(g_hbm.at[srcv.at[0]], buf_a, sem_a)
    pltpu.async_copy(g_hbm.at[srcv.at[1]], buf_b, sem_b)

    def body(i, carry):
        a = 2 * i
        b = a + 1
        pltpu.make_async_copy(g_hbm.at[srcv.at[a]], buf_a, sem_a).wait()
        _scale_rows(buf_a, ewv, a)
        pltpu.sync_copy(buf_a, acc.at[dstv.at[a]], add=True)

        @pl.when(i < RPW // 2 - 1)
        def _():
            pltpu.async_copy(g_hbm.at[srcv.at[a + 2]], buf_a, sem_a)

        pltpu.make_async_copy(g_hbm.at[srcv.at[b]], buf_b, sem_b).wait()
        _scale_rows(buf_b, ewv, b)
        pltpu.sync_copy(buf_b, acc.at[dstv.at[b]], add=True)

        @pl.when(i < RPW // 2 - 1)
        def _():
            pltpu.async_copy(g_hbm.at[srcv.at[b + 2]], buf_b, sem_b)

        return carry

    lax.fori_loop(0, RPW // 2, body, 0)
    plsc.subcore_barrier()
    pltpu.sync_copy(acc.at[pl.ds(sid * 640, 640)],
                    out_hbm.at[cid, pl.ds(sid * 640, 640)])


def _head_body(part_ref, g_ref, deg_ref, bg_ref, gam_ref, bet_ref,
               w1t_ref, b1_ref, w2t_ref, b2_ref, out_ref):
    dsum = deg_ref[0] + deg_ref[1] + 1.0
    dinv = jnp.where(dsum > 0, lax.rsqrt(dsum), 0.0)[:N]
    p = part_ref[0, :N, :] + part_ref[1, :N, :]
    agg = dinv * (p + g_ref[...])
    t = jnp.maximum(agg + bg_ref[...], 0.0)
    mu = jnp.mean(t, axis=1, keepdims=True)
    d = t - mu
    var = jnp.mean(d * d, axis=1, keepdims=True)
    ln = d * lax.rsqrt(var + 1e-5) * gam_ref[...] + bet_ref[...]
    pooled = jnp.sum(ln, axis=0, keepdims=True) * (1.0 / N)
    z = jnp.maximum(
        jnp.dot(pooled, w1t_ref[...], preferred_element_type=jnp.float32)
        + b1_ref[...], 0.0)
    out_ref[...] = (jnp.dot(z, w2t_ref[...], preferred_element_type=jnp.float32)
                    + b2_ref[...])


def _make_head(a_dim):
    return pl.pallas_call(
        _head_body,
        out_shape=jax.ShapeDtypeStruct((1, a_dim), jnp.float32),
    )


def kernel(x, edge_index, edge_weight, W_gcn, b_gcn, gamma, beta, W1, b1, W2, b2):
    src2 = edge_index[0].reshape(NW, RPW, ROW)
    dst2 = edge_index[1].reshape(NW, RPW, ROW)
    ew2 = edge_weight.reshape(NW, RPW, ROW)

    deg_p = _deg_kernel(dst2, ew2)
    deg3 = deg_p.reshape(NC, NPAD, 1)
    g = _g_call(x, W_gcn.T, deg3)
    part = _agg_kernel(src2, dst2, ew2, g)
    head = _make_head(W2.shape[0])
    return head(part, g, deg3, b_gcn.reshape(1, H1), gamma.reshape(1, H1),
                beta.reshape(1, H1), W1.T, b1.reshape(1, -1), W2.T,
                b2.reshape(1, -1))


# speedup vs baseline: 43.2868x; 1.0063x over previous
import functools

import jax
import jax.numpy as jnp
from jax import lax
from jax.experimental import pallas as pl
from jax.experimental.pallas import tpu as pltpu
from jax.experimental.pallas import tpu_sc as plsc

N = 10000
E = 320000
D_IN = 128
H1 = 64
ROW = 100
NC = 2
NS = 16
NW = NC * NS
RPW = E // (NW * ROW)
NPAD = 10240

_sc_mesh = plsc.VectorSubcoreMesh(core_axis_name="c", subcore_axis_name="s")
_sc_params = pltpu.CompilerParams(use_tc_tiling_on_sc=False)


@functools.partial(
    pl.kernel,
    out_type=jax.ShapeDtypeStruct((NC, NPAD), jnp.float32),
    mesh=_sc_mesh,
    compiler_params=_sc_params,
    scratch_types=[
        pltpu.VMEM((640,), jnp.float32),
        pltpu.VMEM((RPW, ROW), jnp.int32),
        pltpu.VMEM((RPW, ROW), jnp.float32),
        pltpu.VMEM_SHARED((NPAD,), jnp.float32),
        pltpu.SemaphoreType.DMA,
    ],
)
def _deg_kernel(dst_hbm, ew_hbm, out_hbm, zbuf, dstv, ewv, acc, sem):
    cid = lax.axis_index("c")
    sid = lax.axis_index("s")
    wid = sid * NC + cid
    zeros = jnp.zeros((16,), jnp.float32)
    for i in range(40):
        zbuf[pl.ds(i * 16, 16)] = zeros
    pltpu.sync_copy(zbuf, acc.at[pl.ds(sid * 640, 640)])
    plsc.subcore_barrier()
    pltpu.sync_copy(dst_hbm.at[wid], dstv)
    pltpu.sync_copy(ew_hbm.at[wid], ewv)

    def fire(j, carry):
        pltpu.async_copy(ewv.at[j], acc.at[dstv.at[j]], sem, add=True)
        return carry

    lax.fori_loop(0, RPW, fire, 0)

    def drain(j, carry):
        pltpu.make_async_copy(ewv.at[j], acc.at[dstv.at[j]], sem).wait()
        return carry

    lax.fori_loop(0, RPW, drain, 0)
    plsc.subcore_barrier()

    @pl.when(sid == 0)
    def _():
        pltpu.sync_copy(acc, out_hbm.at[cid])


def _g_body(x_ref, wt_ref, deg_ref, g_ref):
    dsum = deg_ref[0] + deg_ref[1] + 1.0
    dinv = jnp.where(dsum > 0, lax.rsqrt(dsum), 0.0)
    h = jnp.dot(x_ref[...], wt_ref[...], preferred_element_type=jnp.float32)
    g_ref[...] = dinv[:N] * h


_g_call = pl.pallas_call(
    _g_body,
    out_shape=jax.ShapeDtypeStruct((N, H1), jnp.float32),
)


def _scale_rows(buf, ewv, j):
    offs = [0, 16, 32, 48, 64, 80]
    for grp, off in enumerate(offs):
        wv = ewv[j, pl.ds(off, 16)]
        for e16 in range(16):
            e = off + e16
            w = wv[e16]
            for k in range(H1 // 16):
                sl = pl.ds(k * 16, 16)
                buf[e, sl] = buf[e, sl] * w
    wv = ewv[j, pl.ds(84, 16)]
    for e in range(96, ROW):
        w = wv[e - 84]
        for k in range(H1 // 16):
            sl = pl.ds(k * 16, 16)
            buf[e, sl] = buf[e, sl] * w


@functools.partial(
    pl.kernel,
    out_type=jax.ShapeDtypeStruct((NC, NPAD, H1), jnp.float32),
    mesh=_sc_mesh,
    compiler_params=_sc_params,
    scratch_types=[
        pltpu.VMEM((64, H1), jnp.float32),
        pltpu.VMEM((RPW, ROW), jnp.int32),
        pltpu.VMEM((RPW, ROW), jnp.int32),
        pltpu.VMEM((RPW, ROW), jnp.float32),
        pltpu.VMEM((ROW, H1), jnp.float32),
        pltpu.VMEM((ROW, H1), jnp.float32),
        pltpu.VMEM((ROW, H1), jnp.float32),
        pltpu.VMEM((ROW, H1), jnp.float32),
        pltpu.VMEM_SHARED((NPAD, H1), jnp.float32),
        pltpu.SemaphoreType.DMA,
        pltpu.SemaphoreType.DMA,
        pltpu.SemaphoreType.DMA,
        pltpu.SemaphoreType.DMA,
        pltpu.SemaphoreType.DMA,
        pltpu.SemaphoreType.DMA,
        pltpu.SemaphoreType.DMA,
        pltpu.SemaphoreType.DMA,
    ],
)
def _agg_kernel(src_hbm, dst_hbm, ew_hbm, g_hbm, out_hbm,
                zb, srcv, dstv, ewv, b0, b1, b2, b3, acc,
                sg0, sg1, sg2, sg3, ss0, ss1, ss2, ss3):
    cid = lax.axis_index("c")
    sid = lax.axis_index("s")
    wid = sid * NC + cid
    zeros = jnp.zeros((16,), jnp.float32)
    for r in range(64):
        for k in range(H1 // 16):
            zb[r, pl.ds(k * 16, 16)] = zeros
    for k in range(10):
        pltpu.sync_copy(zb, acc.at[pl.ds(sid * 640 + k * 64, 64)])
    plsc.subcore_barrier()
    pltpu.sync_copy(src_hbm.at[wid], srcv)
    pltpu.sync_copy(dst_hbm.at[wid], dstv)
    pltpu.sync_copy(ew_hbm.at[wid], ewv)

    bufs = [b0, b1, b2, b3]
    sgs = [sg0, sg1, sg2, sg3]
    sss = [ss0, ss1, ss2, ss3]
    NIT = RPW // 4

    for k in range(3):
        pltpu.async_copy(g_hbm.at[srcv.at[k]], bufs[k], sgs[k])

    def body(i, carry):
        for k in range(4):
            r = 4 * i + k
            pltpu.make_async_copy(g_hbm.at[srcv.at[r]], bufs[k], sgs[k]).wait()
            _scale_rows(bufs[k], ewv, r)
            pltpu.async_copy(bufs[k], acc.at[dstv.at[r]], sss[k], add=True)
            q = (k + 3) % 4
            nxt = r + 3
            if k == 0:
                @pl.when(i > 0)
                def _():
                    pltpu.make_async_copy(bufs[q], acc.at[dstv.at[0]],
                                          sss[q]).wait()
                pltpu.async_copy(g_hbm.at[srcv.at[nxt]], bufs[q], sgs[q])
            else:
                pltpu.make_async_copy(bufs[q], acc.at[dstv.at[0]],
                                      sss[q]).wait()

                @pl.when(i < NIT - 1)
                def _():
                    pltpu.async_copy(g_hbm.at[srcv.at[nxt]], bufs[q], sgs[q])
        return carry

    lax.fori_loop(0, NIT, body, 0)
    pltpu.make_async_copy(bufs[3], acc.at[dstv.at[0]], sss[3]).wait()
    plsc.subcore_barrier()
    pltpu.sync_copy(acc.at[pl.ds(sid * 640, 640)],
                    out_hbm.at[cid, pl.ds(sid * 640, 640)])


def _head_body(part_ref, g_ref, deg_ref, bg_ref, gam_ref, bet_ref,
               w1t_ref, b1_ref, w2t_ref, b2_ref, out_ref):
    dsum = deg_ref[0] + deg_ref[1] + 1.0
    dinv = jnp.where(dsum > 0, lax.rsqrt(dsum), 0.0)[:N]
    p = part_ref[0, :N, :] + part_ref[1, :N, :]
    agg = dinv * (p + g_ref[...])
    t = jnp.maximum(agg + bg_ref[...], 0.0)
    mu = jnp.mean(t, axis=1, keepdims=True)
    d = t - mu
    var = jnp.mean(d * d, axis=1, keepdims=True)
    ln = d * lax.rsqrt(var + 1e-5) * gam_ref[...] + bet_ref[...]
    pooled = jnp.sum(ln, axis=0, keepdims=True) * (1.0 / N)
    z = jnp.maximum(
        jnp.dot(pooled, w1t_ref[...], preferred_element_type=jnp.float32)
        + b1_ref[...], 0.0)
    out_ref[...] = (jnp.dot(z, w2t_ref[...], preferred_element_type=jnp.float32)
                    + b2_ref[...])


def _make_head(a_dim):
    return pl.pallas_call(
        _head_body,
        out_shape=jax.ShapeDtypeStruct((1, a_dim), jnp.float32),
    )


def kernel(x, edge_index, edge_weight, W_gcn, b_gcn, gamma, beta, W1, b1, W2, b2):
    src2 = edge_index[0].reshape(NW, RPW, ROW)
    dst2 = edge_index[1].reshape(NW, RPW, ROW)
    ew2 = edge_weight.reshape(NW, RPW, ROW)

    deg_p = _deg_kernel(dst2, ew2)
    deg3 = deg_p.reshape(NC, NPAD, 1)
    g = _g_call(x, W_gcn.T, deg3)
    part = _agg_kernel(src2, dst2, ew2, g)
    head = _make_head(W2.shape[0])
    return head(part, g, deg3, b_gcn.reshape(1, H1), gamma.reshape(1, H1),
                beta.reshape(1, H1), W1.T, b1.reshape(1, -1), W2.T,
                b2.reshape(1, -1))
